# baseline (speedup 1.0000x reference)
import jax
import jax.numpy as jnp
from jax.experimental import pallas as pl
from jax.experimental.pallas import tpu as pltpu

MIN_MZ, MAX_MZ, BIN_W = 0.0, 1000.0, 0.01
NUM_BINS = int((MAX_MZ - MIN_MZ) / BIN_W)
GROUP, OPG = 30, 3
GROUPS = NUM_BINS // GROUP
SCALING = 0.5
EPS = 1e-6

P = 512
G_PAD = 3456
I_PAD = 32
SPEC_PER_STEP = 16
H1 = 1000
GB = 128


def _bin_kernel(mz_ref, it_ref, wr_ref, bb_ref, o_ref, g_scr, v_scr):
    mzr = mz_ref[...]
    itr = it_ref[...]
    maskr = (mzr >= MIN_MZ) & (mzr < MAX_MZ)
    idxr = jnp.clip(((mzr - MIN_MZ) / BIN_W).astype(jnp.int32), 0, NUM_BINS - 1)
    valr = jnp.where(maskr & (idxr < GROUPS * GROUP), jnp.sqrt(itr), 0.0)
    gr = idxr // GROUP
    iir16 = (idxr - gr * GROUP).astype(jnp.int16)

    g_scr[...] = gr.T.astype(jnp.int16)
    v_scr[...] = valr.T.astype(jnp.bfloat16)

    iota_i = jax.lax.broadcasted_iota(jnp.int16, (I_PAD, P), 0)
    iota_g = jax.lax.broadcasted_iota(jnp.int16, (P, G_PAD), 1)

    def onehot_dot(s):
        a_t = jnp.where(iota_i == iir16[s : s + 1, :], jnp.bfloat16(1.0),
                        jnp.bfloat16(0.0))
        ohg = jnp.where(iota_g == g_scr[:, s : s + 1], v_scr[:, s : s + 1],
                        jnp.bfloat16(0.0))
        return jax.lax.dot_general(
            a_t, ohg, (((1,), (0,)), ((), ())),
            preferred_element_type=jnp.float32)

    def contract(s, bgt):
        for c in range(G_PAD // 128):
            sl = slice(c * 128, (c + 1) * 128)
            b = bgt[:, sl]
            for o in range(OPG):
                xo = jnp.sum(b * wr_ref[o, :, sl], axis=0, keepdims=True) \
                    + bb_ref[o : o + 1, sl]
                o_ref[o : o + 1, s : s + 1, sl] = xo[None].astype(jnp.bfloat16)

    prev = onehot_dot(0)
    for s in range(1, SPEC_PER_STEP):
        cur = onehot_dot(s)
        contract(s - 1, prev)
        prev = cur
    contract(SPEC_PER_STEP - 1, prev)


def _w0perm_kernel(w0_ref, o_ref):
    j = pl.program_id(0)
    v = w0_ref[...]
    nvalid = GROUPS * OPG - j * OPG * GB
    row = jax.lax.broadcasted_iota(jnp.int32, (OPG * GB, H1), 0)
    v = jnp.where(row < nvalid, v, 0.0)
    v3 = v.reshape(GB, OPG, H1)
    for o in range(OPG):
        o_ref[o] = v3[:, o, :].astype(jnp.bfloat16)


def _mlp_kernel(x_ref, w0_ref, b0_ref, w1_ref, b1_ref, w2_ref, b2_ref,
                we_ref, be_ref, o_ref, h1_ref):
    j = pl.program_id(1)
    acc = jax.lax.dot_general(
        x_ref[0], w0_ref[0], (((1,), (0,)), ((), ())),
        preferred_element_type=jnp.float32)
    for o in range(1, OPG):
        acc = acc + jax.lax.dot_general(
            x_ref[o], w0_ref[o], (((1,), (0,)), ((), ())),
            preferred_element_type=jnp.float32)

    @pl.when(j == 0)
    def _():
        h1_ref[...] = acc

    @pl.when(j == 1)
    def _():
        h1_ref[...] = h1_ref[...] + acc

    @pl.when(j == 2)
    def _():
        h1 = h1_ref[...] + acc + b0_ref[...]
        h1 = jnp.maximum(h1, 0.0).astype(jnp.bfloat16)
        h2 = jax.lax.dot_general(
            h1, w1_ref[...].astype(jnp.bfloat16), (((1,), (0,)), ((), ())),
            preferred_element_type=jnp.float32) + b1_ref[...]
        h2 = jnp.maximum(h2, 0.0).astype(jnp.bfloat16)
        h3 = jax.lax.dot_general(
            h2, w2_ref[...].astype(jnp.bfloat16), (((1,), (0,)), ((), ())),
            preferred_element_type=jnp.float32) + b2_ref[...]
        h3 = jnp.maximum(h3, 0.0).astype(jnp.bfloat16)
        emb = jax.lax.dot_general(
            h3, we_ref[...].astype(jnp.bfloat16), (((1,), (0,)), ((), ())),
            preferred_element_type=jnp.float32) + be_ref[...]
        p12 = emb * pltpu.roll(emb, emb.shape[0] - 1, axis=0)
        s12 = jnp.sum(p12, axis=1, keepdims=True)
        ss = jnp.sum(emb * emb, axis=1, keepdims=True)
        na = jnp.maximum(jnp.sqrt(ss), EPS)
        nb = pltpu.roll(na, na.shape[0] - 1, axis=0)
        o_ref[...] = (s12 / (na * nb))[None]


def kernel(mz, intensities, binner_w, binner_b, w0, b0, w1, b1, w2, b2, we, be):
    bp = mz.shape[0]
    n = bp * 2
    half = bp // 2

    mz2 = mz.reshape(n, P)
    it2 = intensities.reshape(n, P)

    wr = jnp.transpose(binner_w, (2, 1, 0))
    wr = jnp.pad(wr, ((0, 0), (0, I_PAD - GROUP), (0, G_PAD - GROUPS)))
    bb = jnp.transpose(binner_b, (1, 0))
    bb = jnp.pad(bb, ((0, 5), (0, G_PAD - GROUPS)))

    x3 = pl.pallas_call(
        _bin_kernel,
        grid=(n // SPEC_PER_STEP,),
        in_specs=[
            pl.BlockSpec((SPEC_PER_STEP, P), lambda i: (i, 0)),
            pl.BlockSpec((SPEC_PER_STEP, P), lambda i: (i, 0)),
            pl.BlockSpec((OPG, I_PAD, G_PAD), lambda i: (0, 0, 0)),
            pl.BlockSpec((8, G_PAD), lambda i: (0, 0)),
        ],
        out_specs=pl.BlockSpec((OPG, SPEC_PER_STEP, G_PAD), lambda i: (0, i, 0)),
        out_shape=jax.ShapeDtypeStruct((OPG, n, G_PAD), jnp.bfloat16),
        scratch_shapes=[
            pltpu.VMEM((P, SPEC_PER_STEP), jnp.int16),
            pltpu.VMEM((P, SPEC_PER_STEP), jnp.bfloat16),
        ],
        compiler_params=pltpu.CompilerParams(
            dimension_semantics=("arbitrary",),
            vmem_limit_bytes=56 * 1024 * 1024,
        ),
        name="bin_onehot",
    )(mz2, it2, wr, bb)

    w0e = pl.pallas_call(
        _w0perm_kernel,
        grid=(G_PAD // GB,),
        in_specs=[pl.BlockSpec((OPG * GB, H1), lambda j: (j, 0))],
        out_specs=pl.BlockSpec((OPG, GB, H1), lambda j: (0, j, 0)),
        out_shape=jax.ShapeDtypeStruct((OPG, G_PAD, H1), jnp.bfloat16),
        compiler_params=pltpu.CompilerParams(
            dimension_semantics=("arbitrary",),
            vmem_limit_bytes=56 * 1024 * 1024,
        ),
        name="w0_permute",
    )(w0)

    b0r = b0.reshape(1, H1)
    b1r = b1.reshape(1, 800)
    b2r = b2.reshape(1, 800)
    ber = be.reshape(1, 400)

    KH = G_PAD // 3
    out = pl.pallas_call(
        _mlp_kernel,
        grid=(2, 3),
        in_specs=[
            pl.BlockSpec((OPG, n // 2, KH), lambda h, j: (0, h, j)),
            pl.BlockSpec((OPG, KH, H1), lambda h, j: (0, j, 0)),
            pl.BlockSpec((1, H1), lambda h, j: (0, 0)),
            pl.BlockSpec((H1, 800), lambda h, j: (0, 0)),
            pl.BlockSpec((1, 800), lambda h, j: (0, 0)),
            pl.BlockSpec((800, 800), lambda h, j: (0, 0)),
            pl.BlockSpec((1, 800), lambda h, j: (0, 0)),
            pl.BlockSpec((800, 400), lambda h, j: (0, 0)),
            pl.BlockSpec((1, 400), lambda h, j: (0, 0)),
        ],
        out_specs=pl.BlockSpec((1, n // 2, 1), lambda h, j: (h, 0, 0)),
        out_shape=jax.ShapeDtypeStruct((2, n // 2, 1), jnp.float32),
        scratch_shapes=[pltpu.VMEM((n // 2, H1), jnp.float32)],
        compiler_params=pltpu.CompilerParams(
            dimension_semantics=("arbitrary", "arbitrary"),
            vmem_limit_bytes=56 * 1024 * 1024,
        ),
        name="mlp_cosine",
    )(x3, w0e, b0r, w1, b1r, w2, b2r, we, ber)

    return out.reshape(n)[0::2]


# speedup vs baseline: 1.0150x; 1.0150x over previous
import jax
import jax.numpy as jnp
from jax.experimental import pallas as pl
from jax.experimental.pallas import tpu as pltpu

MIN_MZ, MAX_MZ, BIN_W = 0.0, 1000.0, 0.01
NUM_BINS = int((MAX_MZ - MIN_MZ) / BIN_W)
GROUP, OPG = 30, 3
GROUPS = NUM_BINS // GROUP
SCALING = 0.5
EPS = 1e-6

P = 512
G_PAD = 3456
I_PAD = 32
SPEC_PER_STEP = 32
H1 = 1000
GB = 384


def _bin_kernel(mz_ref, it_ref, wr_ref, bb_ref, o_ref, g_scr, v_scr):
    mzr = mz_ref[...]
    itr = it_ref[...]
    maskr = (mzr >= MIN_MZ) & (mzr < MAX_MZ)
    idxr = jnp.clip(((mzr - MIN_MZ) / BIN_W).astype(jnp.int32), 0, NUM_BINS - 1)
    valr = jnp.where(maskr & (idxr < GROUPS * GROUP), jnp.sqrt(itr), 0.0)
    gr = idxr // GROUP
    iir16 = (idxr - gr * GROUP).astype(jnp.int16)

    g_scr[...] = gr.T.astype(jnp.int16)
    v_scr[...] = valr.T.astype(jnp.bfloat16)

    iota_i = jax.lax.broadcasted_iota(jnp.int16, (I_PAD, P), 0)
    iota_g = jax.lax.broadcasted_iota(jnp.int16, (P, G_PAD), 1)

    def onehot_dot(s):
        a_t = jnp.where(iota_i == iir16[s : s + 1, :], jnp.bfloat16(1.0),
                        jnp.bfloat16(0.0))
        ohg = jnp.where(iota_g == g_scr[:, s : s + 1], v_scr[:, s : s + 1],
                        jnp.bfloat16(0.0))
        return jax.lax.dot_general(
            a_t, ohg, (((1,), (0,)), ((), ())),
            preferred_element_type=jnp.float32)

    def contract(s, bgt):
        for c in range(G_PAD // 128):
            sl = slice(c * 128, (c + 1) * 128)
            b = bgt[:, sl]
            for o in range(OPG):
                xo = jnp.sum(b * wr_ref[o, :, sl], axis=0, keepdims=True) \
                    + bb_ref[o : o + 1, sl]
                o_ref[o : o + 1, s : s + 1, sl] = xo[None].astype(jnp.bfloat16)

    prev = onehot_dot(0)
    for s in range(1, SPEC_PER_STEP):
        cur = onehot_dot(s)
        contract(s - 1, prev)
        prev = cur
    contract(SPEC_PER_STEP - 1, prev)


def _w0perm_kernel(w0_ref, o_ref):
    j = pl.program_id(0)
    v = w0_ref[...]
    nvalid = GROUPS * OPG - j * OPG * GB
    row = jax.lax.broadcasted_iota(jnp.int32, (OPG * GB, H1), 0)
    v = jnp.where(row < nvalid, v, 0.0)
    v3 = v.reshape(GB, OPG, H1)
    for o in range(OPG):
        o_ref[o] = v3[:, o, :].astype(jnp.bfloat16)


def _mlp_kernel(x_ref, w0_ref, b0_ref, w1_ref, b1_ref, w2_ref, b2_ref,
                we_ref, be_ref, o_ref, h1_ref):
    j = pl.program_id(1)
    acc = jax.lax.dot_general(
        x_ref[0], w0_ref[0], (((1,), (0,)), ((), ())),
        preferred_element_type=jnp.float32)
    for o in range(1, OPG):
        acc = acc + jax.lax.dot_general(
            x_ref[o], w0_ref[o], (((1,), (0,)), ((), ())),
            preferred_element_type=jnp.float32)

    @pl.when(j == 0)
    def _():
        h1_ref[...] = acc

    @pl.when(j == 1)
    def _():
        h1_ref[...] = h1_ref[...] + acc

    @pl.when(j == 2)
    def _():
        h1 = h1_ref[...] + acc + b0_ref[...]
        h1 = jnp.maximum(h1, 0.0).astype(jnp.bfloat16)
        h2 = jax.lax.dot_general(
            h1, w1_ref[...].astype(jnp.bfloat16), (((1,), (0,)), ((), ())),
            preferred_element_type=jnp.float32) + b1_ref[...]
        h2 = jnp.maximum(h2, 0.0).astype(jnp.bfloat16)
        h3 = jax.lax.dot_general(
            h2, w2_ref[...].astype(jnp.bfloat16), (((1,), (0,)), ((), ())),
            preferred_element_type=jnp.float32) + b2_ref[...]
        h3 = jnp.maximum(h3, 0.0).astype(jnp.bfloat16)
        emb = jax.lax.dot_general(
            h3, we_ref[...].astype(jnp.bfloat16), (((1,), (0,)), ((), ())),
            preferred_element_type=jnp.float32) + be_ref[...]
        p12 = emb * pltpu.roll(emb, emb.shape[0] - 1, axis=0)
        s12 = jnp.sum(p12, axis=1, keepdims=True)
        ss = jnp.sum(emb * emb, axis=1, keepdims=True)
        na = jnp.maximum(jnp.sqrt(ss), EPS)
        nb = pltpu.roll(na, na.shape[0] - 1, axis=0)
        o_ref[...] = (s12 / (na * nb))[None]


def kernel(mz, intensities, binner_w, binner_b, w0, b0, w1, b1, w2, b2, we, be):
    bp = mz.shape[0]
    n = bp * 2
    half = bp // 2

    mz2 = mz.reshape(n, P)
    it2 = intensities.reshape(n, P)

    wr = jnp.transpose(binner_w, (2, 1, 0))
    wr = jnp.pad(wr, ((0, 0), (0, I_PAD - GROUP), (0, G_PAD - GROUPS)))
    bb = jnp.transpose(binner_b, (1, 0))
    bb = jnp.pad(bb, ((0, 5), (0, G_PAD - GROUPS)))

    x3 = pl.pallas_call(
        _bin_kernel,
        grid=(n // SPEC_PER_STEP,),
        in_specs=[
            pl.BlockSpec((SPEC_PER_STEP, P), lambda i: (i, 0)),
            pl.BlockSpec((SPEC_PER_STEP, P), lambda i: (i, 0)),
            pl.BlockSpec((OPG, I_PAD, G_PAD), lambda i: (0, 0, 0)),
            pl.BlockSpec((8, G_PAD), lambda i: (0, 0)),
        ],
        out_specs=pl.BlockSpec((OPG, SPEC_PER_STEP, G_PAD), lambda i: (0, i, 0)),
        out_shape=jax.ShapeDtypeStruct((OPG, n, G_PAD), jnp.bfloat16),
        scratch_shapes=[
            pltpu.VMEM((P, SPEC_PER_STEP), jnp.int16),
            pltpu.VMEM((P, SPEC_PER_STEP), jnp.bfloat16),
        ],
        compiler_params=pltpu.CompilerParams(
            dimension_semantics=("arbitrary",),
            vmem_limit_bytes=56 * 1024 * 1024,
        ),
        name="bin_onehot",
    )(mz2, it2, wr, bb)

    w0e = pl.pallas_call(
        _w0perm_kernel,
        grid=(G_PAD // GB,),
        in_specs=[pl.BlockSpec((OPG * GB, H1), lambda j: (j, 0))],
        out_specs=pl.BlockSpec((OPG, GB, H1), lambda j: (0, j, 0)),
        out_shape=jax.ShapeDtypeStruct((OPG, G_PAD, H1), jnp.bfloat16),
        compiler_params=pltpu.CompilerParams(
            dimension_semantics=("arbitrary",),
            vmem_limit_bytes=56 * 1024 * 1024,
        ),
        name="w0_permute",
    )(w0)

    b0r = b0.reshape(1, H1)
    b1r = b1.reshape(1, 800)
    b2r = b2.reshape(1, 800)
    ber = be.reshape(1, 400)

    KH = G_PAD // 3
    out = pl.pallas_call(
        _mlp_kernel,
        grid=(2, 3),
        in_specs=[
            pl.BlockSpec((OPG, n // 2, KH), lambda h, j: (0, h, j)),
            pl.BlockSpec((OPG, KH, H1), lambda h, j: (0, j, 0)),
            pl.BlockSpec((1, H1), lambda h, j: (0, 0)),
            pl.BlockSpec((H1, 800), lambda h, j: (0, 0)),
            pl.BlockSpec((1, 800), lambda h, j: (0, 0)),
            pl.BlockSpec((800, 800), lambda h, j: (0, 0)),
            pl.BlockSpec((1, 800), lambda h, j: (0, 0)),
            pl.BlockSpec((800, 400), lambda h, j: (0, 0)),
            pl.BlockSpec((1, 400), lambda h, j: (0, 0)),
        ],
        out_specs=pl.BlockSpec((1, n // 2, 1), lambda h, j: (h, 0, 0)),
        out_shape=jax.ShapeDtypeStruct((2, n // 2, 1), jnp.float32),
        scratch_shapes=[pltpu.VMEM((n // 2, H1), jnp.float32)],
        compiler_params=pltpu.CompilerParams(
            dimension_semantics=("arbitrary", "arbitrary"),
            vmem_limit_bytes=56 * 1024 * 1024,
        ),
        name="mlp_cosine",
    )(x3, w0e, b0r, w1, b1r, w2, b2r, we, ber)

    return out.reshape(n)[0::2]
